# baseline (speedup 1.0000x reference)
import functools

import jax
import jax.numpy as jnp
from jax import lax
from jax.experimental import pallas as pl
from jax.experimental.pallas import tpu as pltpu
from jax.experimental.pallas import tpu_sc as plsc

B = 16384
D = 64
NC = 2
NS = 16
NW = NC * NS
BPW = B // NW
CHUNK = 128
NCHUNK = BPW // CHUNK


def _sc_gather_body(uidx_hbm, iidx_hbm, ug_hbm, ig_hbm, um_hbm, im_hbm,
                    ug_out, ig_out, um_out, im_out,
                    idx_u, idx_i, rows_a, rows_b, sem_a, sem_b):
  wid = lax.axis_index("s") * NC + lax.axis_index("c")
  pltpu.sync_copy(uidx_hbm.at[wid], idx_u)
  pltpu.sync_copy(iidx_hbm.at[wid], idx_i)

  def gather_pair(tab_a, tab_b, out_a, out_b):
    cps = []
    for j in range(NCHUNK):
      cps.append(pltpu.async_copy(tab_a.at[idx_u.at[j]], rows_a.at[j], sem_a))
      cps.append(pltpu.async_copy(tab_b.at[idx_i.at[j]], rows_b.at[j], sem_b))
    for cp in cps:
      cp.wait()
    pltpu.sync_copy(rows_a, out_a.at[wid])
    pltpu.sync_copy(rows_b, out_b.at[wid])

  gather_pair(ug_hbm, ig_hbm, ug_out, ig_out)
  gather_pair(um_hbm, im_hbm, um_out, im_out)


@functools.cache
def _make_sc_gather():
  return pl.kernel(
      _sc_gather_body,
      out_type=[jax.ShapeDtypeStruct((NW, NCHUNK, CHUNK, D), jnp.float32)] * 4,
      mesh=plsc.VectorSubcoreMesh(core_axis_name="c", subcore_axis_name="s"),
      scratch_types=[
          pltpu.VMEM((NCHUNK, CHUNK), jnp.int32),
          pltpu.VMEM((NCHUNK, CHUNK), jnp.int32),
          pltpu.VMEM((NCHUNK, CHUNK, D), jnp.float32),
          pltpu.VMEM((NCHUNK, CHUNK, D), jnp.float32),
          pltpu.SemaphoreType.DMA,
          pltpu.SemaphoreType.DMA,
      ],
      compiler_params=pltpu.CompilerParams(use_tc_tiling_on_sc=False),
  )


def _tc_body(ug_ref, ig_ref, um_ref, im_ref, w0u_ref, w0i_ref, b0_ref,
             w1_ref, b1_ref, w2_ref, b2_ref, wog_ref, woh_ref, bo_ref,
             out_ref):
  g = ug_ref[...] * ig_ref[...]
  h = (
      jax.lax.dot_general(um_ref[...], w0u_ref[...], (((1,), (0,)), ((), ())),
                          preferred_element_type=jnp.float32,
                          precision=jax.lax.Precision.HIGHEST)
      + jax.lax.dot_general(im_ref[...], w0i_ref[...], (((1,), (0,)), ((), ())),
                            preferred_element_type=jnp.float32,
                            precision=jax.lax.Precision.HIGHEST)
      + b0_ref[...]
  )
  h = jnp.maximum(h, 0.0)
  h = jax.lax.dot_general(h, w1_ref[...], (((1,), (0,)), ((), ())),
                          preferred_element_type=jnp.float32,
                          precision=jax.lax.Precision.HIGHEST) + b1_ref[...]
  h = jnp.maximum(h, 0.0)
  h = jax.lax.dot_general(h, w2_ref[...], (((1,), (0,)), ((), ())),
                          preferred_element_type=jnp.float32,
                          precision=jax.lax.Precision.HIGHEST) + b2_ref[...]
  h = jnp.maximum(h, 0.0)
  logit = (
      jnp.sum(g * wog_ref[...], axis=1)
      + jnp.sum(h * woh_ref[...], axis=1)
      + bo_ref[0, 0]
  )
  out_ref[...] = 1.0 / (1.0 + jnp.exp(-logit))


def kernel(user_indices, item_indices, ug_table, ig_table, um_table, im_table,
           w0, b0, w1, b1, w2, b2, wo, bo):
  uidx3 = user_indices.astype(jnp.int32).reshape(NW, NCHUNK, CHUNK)
  iidx3 = item_indices.astype(jnp.int32).reshape(NW, NCHUNK, CHUNK)

  ug_rows, ig_rows, um_rows, im_rows = _make_sc_gather()(
      uidx3, iidx3, ug_table, ig_table, um_table, im_table)
  ug_rows = ug_rows.reshape(B, D)
  ig_rows = ig_rows.reshape(B, D)
  um_rows = um_rows.reshape(B, D)
  im_rows = im_rows.reshape(B, D)

  w0u = w0[:, :D].T
  w0i = w0[:, D:].T
  w1t = w1.T
  w2t = w2.T
  wog = wo[:, :D]
  woh = wo[:, D:]
  b0r = b0.reshape(1, -1)
  b1r = b1.reshape(1, -1)
  b2r = b2.reshape(1, -1)
  bor = bo.reshape(1, 1)

  rows_per_tile = 512
  ntiles = B // rows_per_tile

  full = lambda shape: pl.BlockSpec(shape, lambda i: (0, 0))
  out = pl.pallas_call(
      _tc_body,
      grid=(ntiles,),
      in_specs=[
          pl.BlockSpec((rows_per_tile, D), lambda i: (i, 0)),
          pl.BlockSpec((rows_per_tile, D), lambda i: (i, 0)),
          pl.BlockSpec((rows_per_tile, D), lambda i: (i, 0)),
          pl.BlockSpec((rows_per_tile, D), lambda i: (i, 0)),
          full((D, 128)),
          full((D, 128)),
          full((1, 128)),
          full((128, 64)),
          full((1, 64)),
          full((64, 32)),
          full((1, 32)),
          full((1, D)),
          full((1, 32)),
          full((1, 1)),
      ],
      out_specs=pl.BlockSpec((rows_per_tile,), lambda i: (i,)),
      out_shape=jax.ShapeDtypeStruct((B,), jnp.float32),
  )(ug_rows, ig_rows, um_rows, im_rows, w0u, w0i, b0r, w1t, b1r, w2t, b2r,
    wog, woh, bor)
  return out


# speedup vs baseline: 1.5734x; 1.5734x over previous
import functools

import jax
import jax.numpy as jnp
from jax import lax
from jax.experimental import pallas as pl
from jax.experimental.pallas import tpu as pltpu
from jax.experimental.pallas import tpu_sc as plsc

B = 16384
D = 64
NV = 1000000
NC = 2
NS = 16
NW = NC * NS
BPW = B // NW
CHUNK = 128
NCHUNK = BPW // CHUNK


def _pack_body(a_ref, b_ref, out_ref):
  at = a_ref[...].T
  bt = b_ref[...].T
  out_ref[...] = jnp.concatenate([at, bt], axis=1)


def _pack_tables(aT, bT):
  n = 2048
  grid = pl.cdiv(NV, n)
  return pl.pallas_call(
      _pack_body,
      grid=(grid,),
      in_specs=[
          pl.BlockSpec((D, n), lambda i: (0, i)),
          pl.BlockSpec((D, n), lambda i: (0, i)),
      ],
      out_specs=pl.BlockSpec((n, 2 * D), lambda i: (i, 0)),
      out_shape=jax.ShapeDtypeStruct((NV, 2 * D), jnp.float32),
  )(aT, bT)


def _sc_gather_body(uidx_hbm, iidx_hbm, up_hbm, ip_hbm,
                    u_out, i_out,
                    idx_u, idx_i, buf_a, buf_b, sem_a, sem_b):
  wid = lax.axis_index("s") * NC + lax.axis_index("c")
  pltpu.sync_copy(uidx_hbm.at[wid], idx_u)
  pltpu.sync_copy(iidx_hbm.at[wid], idx_i)

  half = NCHUNK // 2

  def fire(tab, idx, c0, buf, sem):
    for c in range(half):
      pltpu.async_copy(tab.at[idx.at[c0 + c]], buf.at[c], sem)

  def drain(out, c0, buf, sem):
    pltpu.make_async_copy(out.at[wid, pl.ds(c0, half)], buf, sem).wait()

  def writeback(buf, out, c0):
    pltpu.sync_copy(buf, out.at[wid, pl.ds(c0, half)])

  fire(up_hbm, idx_u, 0, buf_a, sem_a)
  drain(u_out, 0, buf_a, sem_a)
  fire(up_hbm, idx_u, half, buf_b, sem_b)
  writeback(buf_a, u_out, 0)
  drain(u_out, half, buf_b, sem_b)
  fire(ip_hbm, idx_i, 0, buf_a, sem_a)
  writeback(buf_b, u_out, half)
  drain(i_out, 0, buf_a, sem_a)
  fire(ip_hbm, idx_i, half, buf_b, sem_b)
  writeback(buf_a, i_out, 0)
  drain(i_out, half, buf_b, sem_b)
  writeback(buf_b, i_out, half)


@functools.cache
def _make_sc_gather():
  return pl.kernel(
      _sc_gather_body,
      out_type=[jax.ShapeDtypeStruct((NW, NCHUNK, CHUNK, 2 * D), jnp.float32)] * 2,
      mesh=plsc.VectorSubcoreMesh(core_axis_name="c", subcore_axis_name="s"),
      scratch_types=[
          pltpu.VMEM((NCHUNK, CHUNK), jnp.int32),
          pltpu.VMEM((NCHUNK, CHUNK), jnp.int32),
          pltpu.VMEM((NCHUNK // 2, CHUNK, 2 * D), jnp.float32),
          pltpu.VMEM((NCHUNK // 2, CHUNK, 2 * D), jnp.float32),
          pltpu.SemaphoreType.DMA,
          pltpu.SemaphoreType.DMA,
      ],
      compiler_params=pltpu.CompilerParams(use_tc_tiling_on_sc=False),
  )


def _tc_body(u_ref, i_ref, w0u_ref, w0i_ref, b0_ref,
             w1_ref, b1_ref, w2_ref, b2_ref, wog_ref, woh_ref, bo_ref,
             out_ref):
  mm = functools.partial(
      jax.lax.dot_general,
      dimension_numbers=(((1,), (0,)), ((), ())),
      preferred_element_type=jnp.float32,
      precision=jax.lax.Precision.HIGHEST,
  )
  u = u_ref[...]
  it = i_ref[...]
  g = u[:, :D] * it[:, :D]
  h = mm(u[:, D:], w0u_ref[...]) + mm(it[:, D:], w0i_ref[...]) + b0_ref[...]
  h = jnp.maximum(h, 0.0)
  h = jnp.maximum(mm(h, w1_ref[...]) + b1_ref[...], 0.0)
  h = jnp.maximum(mm(h, w2_ref[...]) + b2_ref[...], 0.0)
  logit = (
      jnp.sum(g * wog_ref[...], axis=1)
      + jnp.sum(h * woh_ref[...], axis=1)
      + bo_ref[0, 0]
  )
  out_ref[...] = 1.0 / (1.0 + jnp.exp(-logit))


def kernel(user_indices, item_indices, ug_table, ig_table, um_table, im_table,
           w0, b0, w1, b1, w2, b2, wo, bo):
  uidx3 = user_indices.astype(jnp.int32).reshape(NW, NCHUNK, CHUNK)
  iidx3 = item_indices.astype(jnp.int32).reshape(NW, NCHUNK, CHUNK)

  ugT, igT, umT, imT = (t.T for t in (ug_table, ig_table, um_table, im_table))

  user_pack = _pack_tables(ugT, umT)
  item_pack = _pack_tables(igT, imT)

  u_rows, i_rows = _make_sc_gather()(uidx3, iidx3, user_pack, item_pack)
  u_rows = u_rows.reshape(B, 2 * D)
  i_rows = i_rows.reshape(B, 2 * D)

  w0u = w0[:, :D].T
  w0i = w0[:, D:].T
  w1t = w1.T
  w2t = w2.T
  wog = wo[:, :D]
  woh = wo[:, D:]
  b0r = b0.reshape(1, -1)
  b1r = b1.reshape(1, -1)
  b2r = b2.reshape(1, -1)
  bor = bo.reshape(1, 1)

  bt = 512
  ntiles = B // bt
  full = lambda shape: pl.BlockSpec(shape, lambda i: (0, 0))
  out = pl.pallas_call(
      _tc_body,
      grid=(ntiles,),
      in_specs=[
          pl.BlockSpec((bt, 2 * D), lambda i: (i, 0)),
          pl.BlockSpec((bt, 2 * D), lambda i: (i, 0)),
          full((D, 128)),
          full((D, 128)),
          full((1, 128)),
          full((128, 64)),
          full((1, 64)),
          full((64, 32)),
          full((1, 32)),
          full((1, D)),
          full((1, 32)),
          full((1, 1)),
      ],
      out_specs=pl.BlockSpec((bt,), lambda i: (i,)),
      out_shape=jax.ShapeDtypeStruct((B,), jnp.float32),
  )(u_rows, i_rows, w0u, w0i, b0r, w1t, b1r, w2t, b2r, wog, woh, bor)
  return out


# speedup vs baseline: 2.3844x; 1.5154x over previous
import functools

import jax
import jax.numpy as jnp
from jax import lax
from jax.experimental import pallas as pl
from jax.experimental.pallas import tpu as pltpu
from jax.experimental.pallas import tpu_sc as plsc

B = 16384
D = 64
NV = 1000000
NC = 2
NS = 16
NW = NC * NS
BPW = B // NW
CHUNK = 128
NCHUNK = BPW // CHUNK


def _pack_body(a_ref, b_ref, c_ref, d_ref, u_ref, i_ref):
  u_ref[...] = jnp.concatenate([a_ref[...].T, b_ref[...].T], axis=1)
  i_ref[...] = jnp.concatenate([c_ref[...].T, d_ref[...].T], axis=1)


def _pack_tables(ugT, umT, igT, imT):
  n = 8192
  grid = pl.cdiv(NV, n)
  return pl.pallas_call(
      _pack_body,
      grid=(grid,),
      in_specs=[pl.BlockSpec((D, n), lambda i: (0, i))] * 4,
      out_specs=[pl.BlockSpec((n, 2 * D), lambda i: (i, 0))] * 2,
      out_shape=[jax.ShapeDtypeStruct((NV, 2 * D), jnp.float32)] * 2,
  )(ugT, umT, igT, imT)


def _sc_gather_body(uidx_hbm, iidx_hbm, up_hbm, ip_hbm,
                    u_out, i_out,
                    idx_u, idx_i, buf_a, buf_b, sem_a, sem_b):
  wid = lax.axis_index("s") * NC + lax.axis_index("c")
  pltpu.sync_copy(uidx_hbm.at[wid], idx_u)
  pltpu.sync_copy(iidx_hbm.at[wid], idx_i)

  half = NCHUNK // 2

  def fire(tab, idx, c0, buf, sem):
    for c in range(half):
      pltpu.async_copy(tab.at[idx.at[c0 + c]], buf.at[c], sem)

  def drain(out, c0, buf, sem):
    pltpu.make_async_copy(out.at[wid, pl.ds(c0, half)], buf, sem).wait()

  def writeback(buf, out, c0):
    pltpu.sync_copy(buf, out.at[wid, pl.ds(c0, half)])

  fire(up_hbm, idx_u, 0, buf_a, sem_a)
  drain(u_out, 0, buf_a, sem_a)
  fire(up_hbm, idx_u, half, buf_b, sem_b)
  writeback(buf_a, u_out, 0)
  drain(u_out, half, buf_b, sem_b)
  fire(ip_hbm, idx_i, 0, buf_a, sem_a)
  writeback(buf_b, u_out, half)
  drain(i_out, 0, buf_a, sem_a)
  fire(ip_hbm, idx_i, half, buf_b, sem_b)
  writeback(buf_a, i_out, 0)
  drain(i_out, half, buf_b, sem_b)
  writeback(buf_b, i_out, half)


@functools.cache
def _make_sc_gather():
  return pl.kernel(
      _sc_gather_body,
      out_type=[jax.ShapeDtypeStruct((NW, NCHUNK, CHUNK, 2 * D), jnp.float32)] * 2,
      mesh=plsc.VectorSubcoreMesh(core_axis_name="c", subcore_axis_name="s"),
      scratch_types=[
          pltpu.VMEM((NCHUNK, CHUNK), jnp.int32),
          pltpu.VMEM((NCHUNK, CHUNK), jnp.int32),
          pltpu.VMEM((NCHUNK // 2, CHUNK, 2 * D), jnp.float32),
          pltpu.VMEM((NCHUNK // 2, CHUNK, 2 * D), jnp.float32),
          pltpu.SemaphoreType.DMA,
          pltpu.SemaphoreType.DMA,
      ],
      compiler_params=pltpu.CompilerParams(use_tc_tiling_on_sc=False),
  )


def _tc_body(u_ref, i_ref, w0u_ref, w0i_ref, b0_ref,
             w1_ref, b1_ref, w2_ref, b2_ref, wog_ref, woh_ref, bo_ref,
             out_ref):
  mm = functools.partial(
      jax.lax.dot_general,
      dimension_numbers=(((1,), (0,)), ((), ())),
      preferred_element_type=jnp.float32,
      precision=jax.lax.Precision.HIGHEST,
  )
  u = u_ref[...]
  it = i_ref[...]
  g = u[:, :D] * it[:, :D]
  h = mm(u[:, D:], w0u_ref[...]) + mm(it[:, D:], w0i_ref[...]) + b0_ref[...]
  h = jnp.maximum(h, 0.0)
  h = jnp.maximum(mm(h, w1_ref[...]) + b1_ref[...], 0.0)
  h = jnp.maximum(mm(h, w2_ref[...]) + b2_ref[...], 0.0)
  logit = (
      jnp.sum(g * wog_ref[...], axis=1)
      + jnp.sum(h * woh_ref[...], axis=1)
      + bo_ref[0, 0]
  )
  out_ref[...] = 1.0 / (1.0 + jnp.exp(-logit))


def kernel(user_indices, item_indices, ug_table, ig_table, um_table, im_table,
           w0, b0, w1, b1, w2, b2, wo, bo):
  uidx3 = user_indices.astype(jnp.int32).reshape(NW, NCHUNK, CHUNK)
  iidx3 = item_indices.astype(jnp.int32).reshape(NW, NCHUNK, CHUNK)

  ugT, igT, umT, imT = (t.T for t in (ug_table, ig_table, um_table, im_table))

  user_pack, item_pack = _pack_tables(ugT, umT, igT, imT)

  u_rows, i_rows = _make_sc_gather()(uidx3, iidx3, user_pack, item_pack)
  u_rows = u_rows.reshape(B, 2 * D)
  i_rows = i_rows.reshape(B, 2 * D)

  w0u = w0[:, :D].T
  w0i = w0[:, D:].T
  w1t = w1.T
  w2t = w2.T
  wog = wo[:, :D]
  woh = wo[:, D:]
  b0r = b0.reshape(1, -1)
  b1r = b1.reshape(1, -1)
  b2r = b2.reshape(1, -1)
  bor = bo.reshape(1, 1)

  bt = 512
  ntiles = B // bt
  full = lambda shape: pl.BlockSpec(shape, lambda i: (0, 0))
  out = pl.pallas_call(
      _tc_body,
      grid=(ntiles,),
      in_specs=[
          pl.BlockSpec((bt, 2 * D), lambda i: (i, 0)),
          pl.BlockSpec((bt, 2 * D), lambda i: (i, 0)),
          full((D, 128)),
          full((D, 128)),
          full((1, 128)),
          full((128, 64)),
          full((1, 64)),
          full((64, 32)),
          full((1, 32)),
          full((1, D)),
          full((1, 32)),
          full((1, 1)),
      ],
      out_specs=pl.BlockSpec((bt,), lambda i: (i,)),
      out_shape=jax.ShapeDtypeStruct((B,), jnp.float32),
  )(u_rows, i_rows, w0u, w0i, b0r, w1t, b1r, w2t, b2r, wog, woh, bor)
  return out


# speedup vs baseline: 2.9680x; 1.2448x over previous
import functools

import jax
import jax.numpy as jnp
from jax import lax
from jax.experimental import pallas as pl
from jax.experimental.pallas import tpu as pltpu
from jax.experimental.pallas import tpu_sc as plsc

B = 16384
D = 64
NV = 1000000
NC = 2
NS = 16
NW = NC * NS
BPW = B // NW
CHUNK = 128
NCHUNK = BPW // CHUNK


def _pack_body(a_ref, b_ref, c_ref, d_ref, eye_ref, u_ref, i_ref):
  e = eye_ref[...]
  tr = lambda x_ref, y_ref: jax.lax.dot_general(
      jnp.concatenate([x_ref[...], y_ref[...]], axis=0).astype(jnp.bfloat16),
      e, (((0,), (0,)), ((), ())), preferred_element_type=jnp.float32)
  u_ref[...] = tr(a_ref, b_ref)
  i_ref[...] = tr(c_ref, d_ref)


def _pack_tables(ugT, umT, igT, imT):
  n = 8192
  grid = pl.cdiv(NV, n)
  eye = jnp.eye(2 * D, dtype=jnp.bfloat16)
  return pl.pallas_call(
      _pack_body,
      grid=(grid,),
      in_specs=[pl.BlockSpec((D, n), lambda i: (0, i))] * 4
      + [pl.BlockSpec((2 * D, 2 * D), lambda i: (0, 0))],
      out_specs=[pl.BlockSpec((n, 2 * D), lambda i: (i, 0))] * 2,
      out_shape=[jax.ShapeDtypeStruct((NV, 2 * D), jnp.float32)] * 2,
  )(ugT, umT, igT, imT, eye)


def _sc_gather_body(uidx_hbm, iidx_hbm, up_hbm, ip_hbm,
                    u_out, i_out,
                    idx_u, idx_i, buf_a, buf_b, sem_a, sem_b):
  wid = lax.axis_index("s") * NC + lax.axis_index("c")
  pltpu.sync_copy(uidx_hbm.at[wid], idx_u)
  pltpu.sync_copy(iidx_hbm.at[wid], idx_i)

  half = NCHUNK // 2

  def fire(tab, idx, c0, buf, sem):
    for c in range(half):
      pltpu.async_copy(tab.at[idx.at[c0 + c]], buf.at[c], sem)

  def drain(out, c0, buf, sem):
    pltpu.make_async_copy(out.at[wid, pl.ds(c0, half)], buf, sem).wait()

  def writeback(buf, out, c0):
    pltpu.sync_copy(buf, out.at[wid, pl.ds(c0, half)])

  fire(up_hbm, idx_u, 0, buf_a, sem_a)
  drain(u_out, 0, buf_a, sem_a)
  fire(up_hbm, idx_u, half, buf_b, sem_b)
  writeback(buf_a, u_out, 0)
  drain(u_out, half, buf_b, sem_b)
  fire(ip_hbm, idx_i, 0, buf_a, sem_a)
  writeback(buf_b, u_out, half)
  drain(i_out, 0, buf_a, sem_a)
  fire(ip_hbm, idx_i, half, buf_b, sem_b)
  writeback(buf_a, i_out, 0)
  drain(i_out, half, buf_b, sem_b)
  writeback(buf_b, i_out, half)


@functools.cache
def _make_sc_gather():
  return pl.kernel(
      _sc_gather_body,
      out_type=[jax.ShapeDtypeStruct((NW, NCHUNK, CHUNK, 2 * D), jnp.float32)] * 2,
      mesh=plsc.VectorSubcoreMesh(core_axis_name="c", subcore_axis_name="s"),
      scratch_types=[
          pltpu.VMEM((NCHUNK, CHUNK), jnp.int32),
          pltpu.VMEM((NCHUNK, CHUNK), jnp.int32),
          pltpu.VMEM((NCHUNK // 2, CHUNK, 2 * D), jnp.float32),
          pltpu.VMEM((NCHUNK // 2, CHUNK, 2 * D), jnp.float32),
          pltpu.SemaphoreType.DMA,
          pltpu.SemaphoreType.DMA,
      ],
      compiler_params=pltpu.CompilerParams(use_tc_tiling_on_sc=False),
  )


def _tc_body(u_ref, i_ref, w0u_ref, w0i_ref, b0_ref,
             w1_ref, b1_ref, w2_ref, b2_ref, wog_ref, woh_ref, bo_ref,
             out_ref):
  mm = functools.partial(
      jax.lax.dot_general,
      dimension_numbers=(((1,), (0,)), ((), ())),
      preferred_element_type=jnp.float32,
      precision=jax.lax.Precision.HIGHEST,
  )
  u = u_ref[...]
  it = i_ref[...]
  g = u[:, :D] * it[:, :D]
  h = mm(u[:, D:], w0u_ref[...]) + mm(it[:, D:], w0i_ref[...]) + b0_ref[...]
  h = jnp.maximum(h, 0.0)
  h = jnp.maximum(mm(h, w1_ref[...]) + b1_ref[...], 0.0)
  h = jnp.maximum(mm(h, w2_ref[...]) + b2_ref[...], 0.0)
  logit = (
      jnp.sum(g * wog_ref[...], axis=1)
      + jnp.sum(h * woh_ref[...], axis=1)
      + bo_ref[0, 0]
  )
  out_ref[...] = 1.0 / (1.0 + jnp.exp(-logit))


def kernel(user_indices, item_indices, ug_table, ig_table, um_table, im_table,
           w0, b0, w1, b1, w2, b2, wo, bo):
  uidx3 = user_indices.astype(jnp.int32).reshape(NW, NCHUNK, CHUNK)
  iidx3 = item_indices.astype(jnp.int32).reshape(NW, NCHUNK, CHUNK)

  ugT, igT, umT, imT = (t.T for t in (ug_table, ig_table, um_table, im_table))

  user_pack, item_pack = _pack_tables(ugT, umT, igT, imT)

  u_rows, i_rows = _make_sc_gather()(uidx3, iidx3, user_pack, item_pack)
  u_rows = u_rows.reshape(B, 2 * D)
  i_rows = i_rows.reshape(B, 2 * D)

  w0u = w0[:, :D].T
  w0i = w0[:, D:].T
  w1t = w1.T
  w2t = w2.T
  wog = wo[:, :D]
  woh = wo[:, D:]
  b0r = b0.reshape(1, -1)
  b1r = b1.reshape(1, -1)
  b2r = b2.reshape(1, -1)
  bor = bo.reshape(1, 1)

  bt = 512
  ntiles = B // bt
  full = lambda shape: pl.BlockSpec(shape, lambda i: (0, 0))
  out = pl.pallas_call(
      _tc_body,
      grid=(ntiles,),
      in_specs=[
          pl.BlockSpec((bt, 2 * D), lambda i: (i, 0)),
          pl.BlockSpec((bt, 2 * D), lambda i: (i, 0)),
          full((D, 128)),
          full((D, 128)),
          full((1, 128)),
          full((128, 64)),
          full((1, 64)),
          full((64, 32)),
          full((1, 32)),
          full((1, D)),
          full((1, 32)),
          full((1, 1)),
      ],
      out_specs=pl.BlockSpec((bt,), lambda i: (i,)),
      out_shape=jax.ShapeDtypeStruct((B,), jnp.float32),
  )(u_rows, i_rows, w0u, w0i, b0r, w1t, b1r, w2t, b2r, wog, woh, bor)
  return out


# speedup vs baseline: 3.7486x; 1.2630x over previous
import functools

import jax
import jax.numpy as jnp
from jax import lax
from jax.experimental import pallas as pl
from jax.experimental.pallas import tpu as pltpu
from jax.experimental.pallas import tpu_sc as plsc

B = 16384
D = 64
NV = 1000000
NC = 2
NS = 16
NW = NC * NS
BPW = B // NW
CHUNK = 128
NCHUNK = BPW // CHUNK


PACK_N = 8192
PACK_N2 = PACK_N // 2
PACK_GRID = (NV + PACK_N - 1) // PACK_N
NP = PACK_GRID * PACK_N


def _pack_body(a_ref, b_ref, c_ref, d_ref, eye_ref, u_ref, i_ref):
  e = eye_ref[...]

  def tr(x_ref, y_ref, out_ref):
    p32 = jax.lax.dot_general(
        jnp.concatenate([x_ref[...], y_ref[...]], axis=0).astype(jnp.bfloat16),
        e, (((0,), (0,)), ((), ())), preferred_element_type=jnp.float32)
    out_ref[...] = pltpu.bitcast(p32.astype(jnp.bfloat16), jnp.float32)

  tr(a_ref, b_ref, u_ref)
  tr(c_ref, d_ref, i_ref)


def _pack_tables(ugT, umT, igT, imT):
  eye = jnp.eye(2 * D, dtype=jnp.bfloat16)
  return pl.pallas_call(
      _pack_body,
      grid=(PACK_GRID,),
      in_specs=[pl.BlockSpec((D, PACK_N), lambda i: (0, i))] * 4
      + [pl.BlockSpec((2 * D, 2 * D), lambda i: (0, 0))],
      out_specs=[pl.BlockSpec((PACK_N2, 2 * D), lambda i: (i, 0))] * 2,
      out_shape=[jax.ShapeDtypeStruct((NP // 2, 2 * D), jnp.float32)] * 2,
  )(ugT, umT, igT, imT, eye)


def _sc_gather_body(uidx_hbm, iidx_hbm, up_hbm, ip_hbm,
                    u_out, i_out,
                    idx_u, idx_i, buf_a, buf_b, sem_a, sem_b):
  wid = lax.axis_index("s") * NC + lax.axis_index("c")
  pltpu.sync_copy(uidx_hbm.at[wid], idx_u)
  pltpu.sync_copy(iidx_hbm.at[wid], idx_i)

  half = NCHUNK // 2

  def fire(tab, idx, c0, buf, sem):
    for c in range(half):
      pltpu.async_copy(tab.at[idx.at[c0 + c]], buf.at[c], sem)

  def drain(out, c0, buf, sem):
    pltpu.make_async_copy(out.at[wid, pl.ds(c0, half)], buf, sem).wait()

  def writeback(buf, out, c0):
    pltpu.sync_copy(buf, out.at[wid, pl.ds(c0, half)])

  fire(up_hbm, idx_u, 0, buf_a, sem_a)
  drain(u_out, 0, buf_a, sem_a)
  fire(up_hbm, idx_u, half, buf_b, sem_b)
  writeback(buf_a, u_out, 0)
  drain(u_out, half, buf_b, sem_b)
  fire(ip_hbm, idx_i, 0, buf_a, sem_a)
  writeback(buf_b, u_out, half)
  drain(i_out, 0, buf_a, sem_a)
  fire(ip_hbm, idx_i, half, buf_b, sem_b)
  writeback(buf_a, i_out, 0)
  drain(i_out, half, buf_b, sem_b)
  writeback(buf_b, i_out, half)


@functools.cache
def _make_sc_gather():
  return pl.kernel(
      _sc_gather_body,
      out_type=[jax.ShapeDtypeStruct((NW, NCHUNK, CHUNK, 2 * D), jnp.float32)] * 2,
      mesh=plsc.VectorSubcoreMesh(core_axis_name="c", subcore_axis_name="s"),
      scratch_types=[
          pltpu.VMEM((NCHUNK, CHUNK), jnp.int32),
          pltpu.VMEM((NCHUNK, CHUNK), jnp.int32),
          pltpu.VMEM((NCHUNK // 2, CHUNK, 2 * D), jnp.float32),
          pltpu.VMEM((NCHUNK // 2, CHUNK, 2 * D), jnp.float32),
          pltpu.SemaphoreType.DMA,
          pltpu.SemaphoreType.DMA,
      ],
      compiler_params=pltpu.CompilerParams(use_tc_tiling_on_sc=False),
  )


def _tc_body(u_ref, i_ref, par_ref, w0u_ref, w0i_ref, b0_ref,
             w1_ref, b1_ref, w2_ref, b2_ref, wog_ref, woh_ref, bo_ref,
             out_ref):
  mm = functools.partial(
      jax.lax.dot_general,
      dimension_numbers=(((1,), (0,)), ((), ())),
      preferred_element_type=jnp.float32,
      precision=jax.lax.Precision.HIGHEST,
  )
  par = par_ref[...].astype(jnp.int32)

  def dec(r, shift):
    w = jax.lax.bitcast_convert_type(r[...], jnp.int32)
    return jax.lax.bitcast_convert_type((w >> shift) << 16, jnp.float32)

  u = dec(u_ref, (par & 1) * 16)
  it = dec(i_ref, ((par >> 1) & 1) * 16)
  g = u[:, :D] * it[:, :D]
  h = mm(u[:, D:], w0u_ref[...]) + mm(it[:, D:], w0i_ref[...]) + b0_ref[...]
  h = jnp.maximum(h, 0.0)
  h = jnp.maximum(mm(h, w1_ref[...]) + b1_ref[...], 0.0)
  h = jnp.maximum(mm(h, w2_ref[...]) + b2_ref[...], 0.0)
  logit = (
      jnp.sum(g * wog_ref[...], axis=1)
      + jnp.sum(h * woh_ref[...], axis=1)
      + bo_ref[0, 0]
  )
  out_ref[...] = 1.0 / (1.0 + jnp.exp(-logit))


def kernel(user_indices, item_indices, ug_table, ig_table, um_table, im_table,
           w0, b0, w1, b1, w2, b2, wo, bo):
  def rrow(ix):
    ix = ix.astype(jnp.int32)
    return (ix // PACK_N) * PACK_N2 + (ix % PACK_N) // 2

  uidx3 = rrow(user_indices).reshape(NW, NCHUNK, CHUNK)
  iidx3 = rrow(item_indices).reshape(NW, NCHUNK, CHUNK)
  par = (user_indices % 2 + 2 * (item_indices % 2)).astype(jnp.int8)
  par = par.reshape(B, 1)

  ugT, igT, umT, imT = (t.T for t in (ug_table, ig_table, um_table, im_table))

  user_pack, item_pack = _pack_tables(ugT, umT, igT, imT)

  u_rows, i_rows = _make_sc_gather()(uidx3, iidx3, user_pack, item_pack)
  u_rows = u_rows.reshape(B, 2 * D)
  i_rows = i_rows.reshape(B, 2 * D)

  w0u = w0[:, :D].T
  w0i = w0[:, D:].T
  w1t = w1.T
  w2t = w2.T
  wog = wo[:, :D]
  woh = wo[:, D:]
  b0r = b0.reshape(1, -1)
  b1r = b1.reshape(1, -1)
  b2r = b2.reshape(1, -1)
  bor = bo.reshape(1, 1)

  bt = 512
  ntiles = B // bt
  full = lambda shape: pl.BlockSpec(shape, lambda i: (0, 0))
  out = pl.pallas_call(
      _tc_body,
      grid=(ntiles,),
      in_specs=[
          pl.BlockSpec((bt, 2 * D), lambda i: (i, 0)),
          pl.BlockSpec((bt, 2 * D), lambda i: (i, 0)),
          pl.BlockSpec((bt, 1), lambda i: (i, 0)),
          full((D, 128)),
          full((D, 128)),
          full((1, 128)),
          full((128, 64)),
          full((1, 64)),
          full((64, 32)),
          full((1, 32)),
          full((1, D)),
          full((1, 32)),
          full((1, 1)),
      ],
      out_specs=pl.BlockSpec((bt,), lambda i: (i,)),
      out_shape=jax.ShapeDtypeStruct((B,), jnp.float32),
  )(u_rows, i_rows, par, w0u, w0i, b0r, w1t, b1r, w2t, b2r, wog, woh, bor)
  return out


# speedup vs baseline: 4.1231x; 1.0999x over previous
import functools

import jax
import jax.numpy as jnp
from jax import lax
from jax.experimental import pallas as pl
from jax.experimental.pallas import tpu as pltpu
from jax.experimental.pallas import tpu_sc as plsc

B = 16384
D = 64
NV = 1000000
NC = 2
NS = 16
NW = NC * NS
BPW = B // NW
CHUNK = 128
NCHUNK = BPW // CHUNK


PACK_N = 8192
PACK_N2 = PACK_N // 2
PACK_GRID = (NV + PACK_N - 1) // PACK_N
NP = PACK_GRID * PACK_N


def _pack_body(a_ref, b_ref, c_ref, d_ref, eye_ref, u_ref, i_ref):
  e = eye_ref[...]

  def tr(x_ref, y_ref, out_ref):
    p32 = jax.lax.dot_general(
        jnp.concatenate([x_ref[...], y_ref[...]], axis=0).astype(jnp.bfloat16),
        e, (((0,), (0,)), ((), ())), preferred_element_type=jnp.float32)
    out_ref[...] = pltpu.bitcast(p32.astype(jnp.bfloat16), jnp.float32)

  tr(a_ref, b_ref, u_ref)
  tr(c_ref, d_ref, i_ref)


def _pack_tables(ugT, umT, igT, imT):
  eye = jnp.eye(2 * D, dtype=jnp.bfloat16)
  return pl.pallas_call(
      _pack_body,
      grid=(PACK_GRID,),
      in_specs=[pl.BlockSpec((D, PACK_N), lambda i: (0, i))] * 4
      + [pl.BlockSpec((2 * D, 2 * D), lambda i: (0, 0))],
      out_specs=[pl.BlockSpec((PACK_N2, 2 * D), lambda i: (i, 0))] * 2,
      out_shape=[jax.ShapeDtypeStruct((NP // 2, 2 * D), jnp.float32)] * 2,
  )(ugT, umT, igT, imT, eye)


def _sc_gather_body(uidx_hbm, iidx_hbm, up_hbm, ip_hbm,
                    u_out, i_out,
                    idx_u, idx_i, buf_a, buf_b, sem_a, sem_b):
  wid = lax.axis_index("s") * NC + lax.axis_index("c")
  pltpu.sync_copy(uidx_hbm.at[wid], idx_u)
  pltpu.sync_copy(iidx_hbm.at[wid], idx_i)

  half = NCHUNK // 2

  def fire(tab, idx, c0, buf, sem):
    for c in range(half):
      pltpu.async_copy(tab.at[idx.at[c0 + c]], buf.at[c], sem)

  def drain(out, c0, buf, sem):
    pltpu.make_async_copy(out.at[wid, pl.ds(c0, half)], buf, sem).wait()

  def writeback(buf, out, c0):
    pltpu.sync_copy(buf, out.at[wid, pl.ds(c0, half)])

  fire(up_hbm, idx_u, 0, buf_a, sem_a)
  drain(u_out, 0, buf_a, sem_a)
  fire(up_hbm, idx_u, half, buf_b, sem_b)
  writeback(buf_a, u_out, 0)
  drain(u_out, half, buf_b, sem_b)
  fire(ip_hbm, idx_i, 0, buf_a, sem_a)
  writeback(buf_b, u_out, half)
  drain(i_out, 0, buf_a, sem_a)
  fire(ip_hbm, idx_i, half, buf_b, sem_b)
  writeback(buf_a, i_out, 0)
  drain(i_out, half, buf_b, sem_b)
  writeback(buf_b, i_out, half)


@functools.cache
def _make_sc_gather():
  return pl.kernel(
      _sc_gather_body,
      out_type=[jax.ShapeDtypeStruct((NW, NCHUNK, CHUNK, 2 * D), jnp.float32)] * 2,
      mesh=plsc.VectorSubcoreMesh(core_axis_name="c", subcore_axis_name="s"),
      scratch_types=[
          pltpu.VMEM((NCHUNK, CHUNK), jnp.int32),
          pltpu.VMEM((NCHUNK, CHUNK), jnp.int32),
          pltpu.VMEM((NCHUNK // 2, CHUNK, 2 * D), jnp.float32),
          pltpu.VMEM((NCHUNK // 2, CHUNK, 2 * D), jnp.float32),
          pltpu.SemaphoreType.DMA,
          pltpu.SemaphoreType.DMA,
      ],
      compiler_params=pltpu.CompilerParams(use_tc_tiling_on_sc=False),
  )


def _tc_body(u_ref, i_ref, par_ref, w0u_ref, w0i_ref, b0_ref,
             w1_ref, b1_ref, w2_ref, b2_ref, wog_ref, woh_ref, bo_ref,
             out_ref):
  mm = lambda x, w: jax.lax.dot_general(
      x.astype(jnp.bfloat16), w.astype(jnp.bfloat16),
      (((1,), (0,)), ((), ())), preferred_element_type=jnp.float32)
  par = par_ref[...].astype(jnp.int32)

  def dec(r, shift):
    w = jax.lax.bitcast_convert_type(r[...], jnp.int32)
    return jax.lax.bitcast_convert_type((w >> shift) << 16, jnp.float32)

  u = dec(u_ref, (par & 1) * 16)
  it = dec(i_ref, ((par >> 1) & 1) * 16)
  g = u[:, :D] * it[:, :D]
  h = mm(u[:, D:], w0u_ref[...]) + mm(it[:, D:], w0i_ref[...]) + b0_ref[...]
  h = jnp.maximum(h, 0.0)
  h = jnp.maximum(mm(h, w1_ref[...]) + b1_ref[...], 0.0)
  h = jnp.maximum(mm(h, w2_ref[...]) + b2_ref[...], 0.0)
  logit = (
      jnp.sum(g * wog_ref[...], axis=1)
      + jnp.sum(h * woh_ref[...], axis=1)
      + bo_ref[0, 0]
  )
  out_ref[...] = 1.0 / (1.0 + jnp.exp(-logit))


def kernel(user_indices, item_indices, ug_table, ig_table, um_table, im_table,
           w0, b0, w1, b1, w2, b2, wo, bo):
  def rrow(ix):
    ix = ix.astype(jnp.int32)
    return (ix // PACK_N) * PACK_N2 + (ix % PACK_N) // 2

  uidx3 = rrow(user_indices).reshape(NW, NCHUNK, CHUNK)
  iidx3 = rrow(item_indices).reshape(NW, NCHUNK, CHUNK)
  par = (user_indices % 2 + 2 * (item_indices % 2)).astype(jnp.int8)
  par = par.reshape(B, 1)

  ugT, igT, umT, imT = (t.T for t in (ug_table, ig_table, um_table, im_table))

  user_pack, item_pack = _pack_tables(ugT, umT, igT, imT)

  u_rows, i_rows = _make_sc_gather()(uidx3, iidx3, user_pack, item_pack)
  u_rows = u_rows.reshape(B, 2 * D)
  i_rows = i_rows.reshape(B, 2 * D)

  w0u = w0[:, :D].T
  w0i = w0[:, D:].T
  w1t = w1.T
  w2t = w2.T
  wog = wo[:, :D]
  woh = wo[:, D:]
  b0r = b0.reshape(1, -1)
  b1r = b1.reshape(1, -1)
  b2r = b2.reshape(1, -1)
  bor = bo.reshape(1, 1)

  bt = 2048
  ntiles = B // bt
  full = lambda shape: pl.BlockSpec(shape, lambda i: (0, 0))
  out = pl.pallas_call(
      _tc_body,
      grid=(ntiles,),
      in_specs=[
          pl.BlockSpec((bt, 2 * D), lambda i: (i, 0)),
          pl.BlockSpec((bt, 2 * D), lambda i: (i, 0)),
          pl.BlockSpec((bt, 1), lambda i: (i, 0)),
          full((D, 128)),
          full((D, 128)),
          full((1, 128)),
          full((128, 64)),
          full((1, 64)),
          full((64, 32)),
          full((1, 32)),
          full((1, D)),
          full((1, 32)),
          full((1, 1)),
      ],
      out_specs=pl.BlockSpec((bt,), lambda i: (i,)),
      out_shape=jax.ShapeDtypeStruct((B,), jnp.float32),
  )(u_rows, i_rows, par, w0u, w0i, b0r, w1t, b1r, w2t, b2r, wog, woh, bor)
  return out


# speedup vs baseline: 4.1849x; 1.0150x over previous
import functools

import jax
import jax.numpy as jnp
from jax import lax
from jax.experimental import pallas as pl
from jax.experimental.pallas import tpu as pltpu
from jax.experimental.pallas import tpu_sc as plsc

B = 16384
D = 64
NV = 1000000
NC = 2
NS = 16
NW = NC * NS
BPW = B // NW
CHUNK = 128
NCHUNK = BPW // CHUNK


PACK_N = 16384
PACK_N2 = PACK_N // 2
PACK_GRID = (NV + PACK_N - 1) // PACK_N
NP = PACK_GRID * PACK_N


def _pack_body(a_ref, b_ref, c_ref, d_ref, eye_ref, u_ref, i_ref):
  e = eye_ref[...]

  def tr(x_ref, y_ref, out_ref):
    p32 = jax.lax.dot_general(
        jnp.concatenate([x_ref[...], y_ref[...]], axis=0).astype(jnp.bfloat16),
        e, (((0,), (0,)), ((), ())), preferred_element_type=jnp.float32)
    out_ref[...] = pltpu.bitcast(p32.astype(jnp.bfloat16), jnp.float32)

  tr(a_ref, b_ref, u_ref)
  tr(c_ref, d_ref, i_ref)


def _pack_tables(ugT, umT, igT, imT):
  eye = jnp.eye(2 * D, dtype=jnp.bfloat16)
  return pl.pallas_call(
      _pack_body,
      grid=(PACK_GRID,),
      in_specs=[pl.BlockSpec((D, PACK_N), lambda i: (0, i))] * 4
      + [pl.BlockSpec((2 * D, 2 * D), lambda i: (0, 0))],
      out_specs=[pl.BlockSpec((PACK_N2, 2 * D), lambda i: (i, 0))] * 2,
      out_shape=[jax.ShapeDtypeStruct((NP // 2, 2 * D), jnp.float32)] * 2,
  )(ugT, umT, igT, imT, eye)


def _sc_gather_body(uidx_hbm, iidx_hbm, up_hbm, ip_hbm,
                    u_out, i_out,
                    idx_u, idx_i, buf_a, buf_b, sem_a, sem_b):
  wid = lax.axis_index("s") * NC + lax.axis_index("c")
  pltpu.sync_copy(uidx_hbm.at[wid], idx_u)
  pltpu.sync_copy(iidx_hbm.at[wid], idx_i)

  half = NCHUNK // 2

  def fire(tab, idx, c0, buf, sem):
    for c in range(half):
      pltpu.async_copy(tab.at[idx.at[c0 + c]], buf.at[c], sem)

  def drain(out, c0, buf, sem):
    pltpu.make_async_copy(out.at[wid, pl.ds(c0, half)], buf, sem).wait()

  def writeback(buf, out, c0):
    pltpu.sync_copy(buf, out.at[wid, pl.ds(c0, half)])

  fire(up_hbm, idx_u, 0, buf_a, sem_a)
  drain(u_out, 0, buf_a, sem_a)
  fire(up_hbm, idx_u, half, buf_b, sem_b)
  writeback(buf_a, u_out, 0)
  drain(u_out, half, buf_b, sem_b)
  fire(ip_hbm, idx_i, 0, buf_a, sem_a)
  writeback(buf_b, u_out, half)
  drain(i_out, 0, buf_a, sem_a)
  fire(ip_hbm, idx_i, half, buf_b, sem_b)
  writeback(buf_a, i_out, 0)
  drain(i_out, half, buf_b, sem_b)
  writeback(buf_b, i_out, half)


@functools.cache
def _make_sc_gather():
  return pl.kernel(
      _sc_gather_body,
      out_type=[jax.ShapeDtypeStruct((NW, NCHUNK, CHUNK, 2 * D), jnp.float32)] * 2,
      mesh=plsc.VectorSubcoreMesh(core_axis_name="c", subcore_axis_name="s"),
      scratch_types=[
          pltpu.VMEM((NCHUNK, CHUNK), jnp.int32),
          pltpu.VMEM((NCHUNK, CHUNK), jnp.int32),
          pltpu.VMEM((NCHUNK // 2, CHUNK, 2 * D), jnp.float32),
          pltpu.VMEM((NCHUNK // 2, CHUNK, 2 * D), jnp.float32),
          pltpu.SemaphoreType.DMA,
          pltpu.SemaphoreType.DMA,
      ],
      compiler_params=pltpu.CompilerParams(use_tc_tiling_on_sc=False),
  )


def _tc_body(u_ref, i_ref, par_ref, w0u_ref, w0i_ref, b0_ref,
             w1_ref, b1_ref, w2_ref, b2_ref, wog_ref, woh_ref, bo_ref,
             out_ref):
  mm = lambda x, w: jax.lax.dot_general(
      x.astype(jnp.bfloat16), w.astype(jnp.bfloat16),
      (((1,), (0,)), ((), ())), preferred_element_type=jnp.float32)
  par = par_ref[...].astype(jnp.int32)

  def dec(r, shift):
    w = jax.lax.bitcast_convert_type(r[...], jnp.int32)
    return jax.lax.bitcast_convert_type((w >> shift) << 16, jnp.float32)

  u = dec(u_ref, (par & 1) * 16)
  it = dec(i_ref, ((par >> 1) & 1) * 16)
  g = u[:, :D] * it[:, :D]
  h = mm(u[:, D:], w0u_ref[...]) + mm(it[:, D:], w0i_ref[...]) + b0_ref[...]
  h = jnp.maximum(h, 0.0)
  h = jnp.maximum(mm(h, w1_ref[...]) + b1_ref[...], 0.0)
  h = jnp.maximum(mm(h, w2_ref[...]) + b2_ref[...], 0.0)
  logit = (
      jnp.sum(g * wog_ref[...], axis=1)
      + jnp.sum(h * woh_ref[...], axis=1)
      + bo_ref[0, 0]
  )
  out_ref[...] = 1.0 / (1.0 + jnp.exp(-logit))


def kernel(user_indices, item_indices, ug_table, ig_table, um_table, im_table,
           w0, b0, w1, b1, w2, b2, wo, bo):
  def rrow(ix):
    ix = ix.astype(jnp.int32)
    return (ix // PACK_N) * PACK_N2 + (ix % PACK_N) // 2

  uidx3 = rrow(user_indices).reshape(NW, NCHUNK, CHUNK)
  iidx3 = rrow(item_indices).reshape(NW, NCHUNK, CHUNK)
  par = (user_indices % 2 + 2 * (item_indices % 2)).astype(jnp.int8)
  par = par.reshape(B, 1)

  ugT, igT, umT, imT = (t.T for t in (ug_table, ig_table, um_table, im_table))

  user_pack, item_pack = _pack_tables(ugT, umT, igT, imT)

  u_rows, i_rows = _make_sc_gather()(uidx3, iidx3, user_pack, item_pack)
  u_rows = u_rows.reshape(B, 2 * D)
  i_rows = i_rows.reshape(B, 2 * D)

  w0u = w0[:, :D].T
  w0i = w0[:, D:].T
  w1t = w1.T
  w2t = w2.T
  wog = wo[:, :D]
  woh = wo[:, D:]
  b0r = b0.reshape(1, -1)
  b1r = b1.reshape(1, -1)
  b2r = b2.reshape(1, -1)
  bor = bo.reshape(1, 1)

  bt = 2048
  ntiles = B // bt
  full = lambda shape: pl.BlockSpec(shape, lambda i: (0, 0))
  out = pl.pallas_call(
      _tc_body,
      grid=(ntiles,),
      in_specs=[
          pl.BlockSpec((bt, 2 * D), lambda i: (i, 0)),
          pl.BlockSpec((bt, 2 * D), lambda i: (i, 0)),
          pl.BlockSpec((bt, 1), lambda i: (i, 0)),
          full((D, 128)),
          full((D, 128)),
          full((1, 128)),
          full((128, 64)),
          full((1, 64)),
          full((64, 32)),
          full((1, 32)),
          full((1, D)),
          full((1, 32)),
          full((1, 1)),
      ],
      out_specs=pl.BlockSpec((bt,), lambda i: (i,)),
      out_shape=jax.ShapeDtypeStruct((B,), jnp.float32),
  )(u_rows, i_rows, par, w0u, w0i, b0r, w1t, b1r, w2t, b2r, wog, woh, bor)
  return out


# speedup vs baseline: 4.1985x; 1.0032x over previous
import functools

import jax
import jax.numpy as jnp
from jax import lax
from jax.experimental import pallas as pl
from jax.experimental.pallas import tpu as pltpu
from jax.experimental.pallas import tpu_sc as plsc

B = 16384
D = 64
NV = 1000000
NC = 2
NS = 16
NW = NC * NS
BPW = B // NW
CHUNK = 128
NCHUNK = BPW // CHUNK


PACK_N = 16384
PACK_N2 = PACK_N // 2
PACK_GRID = (NV + PACK_N - 1) // PACK_N
NP = PACK_GRID * PACK_N


def _pack_body(a_ref, b_ref, c_ref, d_ref, eye_ref, u_ref, i_ref):
  e = eye_ref[...]

  def tr(x_ref, y_ref, out_ref):
    p32 = jax.lax.dot_general(
        jnp.concatenate([x_ref[...], y_ref[...]], axis=0).astype(jnp.bfloat16),
        e, (((0,), (0,)), ((), ())), preferred_element_type=jnp.float32)
    out_ref[...] = pltpu.bitcast(p32.astype(jnp.bfloat16), jnp.float32)

  tr(a_ref, b_ref, u_ref)
  tr(c_ref, d_ref, i_ref)


def _pack_tables(ugT, umT, igT, imT):
  eye = jnp.eye(2 * D, dtype=jnp.bfloat16)
  return pl.pallas_call(
      _pack_body,
      grid=(PACK_GRID,),
      in_specs=[pl.BlockSpec((D, PACK_N), lambda i: (0, i))] * 4
      + [pl.BlockSpec((2 * D, 2 * D), lambda i: (0, 0))],
      out_specs=[pl.BlockSpec((PACK_N2, 2 * D), lambda i: (i, 0))] * 2,
      out_shape=[jax.ShapeDtypeStruct((NP // 2, 2 * D), jnp.float32)] * 2,
  )(ugT, umT, igT, imT, eye)


def _sc_gather_body(uidx_hbm, iidx_hbm, up_hbm, ip_hbm,
                    u_out, i_out,
                    idx_u, idx_i, buf_a, buf_b, sem_a, sem_b):
  wid = lax.axis_index("s") * NC + lax.axis_index("c")
  pltpu.sync_copy(uidx_hbm.at[wid], idx_u)
  pltpu.sync_copy(iidx_hbm.at[wid], idx_i)

  half = NCHUNK // 2

  def fire(tab, idx, c0, buf, sem):
    for c in range(half):
      pltpu.async_copy(tab.at[idx.at[c0 + c]], buf.at[c], sem)

  def drain(out, c0, buf, sem):
    pltpu.make_async_copy(out.at[wid, pl.ds(c0, half)], buf, sem).wait()

  def writeback(buf, out, c0):
    pltpu.sync_copy(buf, out.at[wid, pl.ds(c0, half)])

  fire(up_hbm, idx_u, 0, buf_a, sem_a)
  drain(u_out, 0, buf_a, sem_a)
  fire(up_hbm, idx_u, half, buf_b, sem_b)
  writeback(buf_a, u_out, 0)
  drain(u_out, half, buf_b, sem_b)
  fire(ip_hbm, idx_i, 0, buf_a, sem_a)
  writeback(buf_b, u_out, half)
  drain(i_out, 0, buf_a, sem_a)
  fire(ip_hbm, idx_i, half, buf_b, sem_b)
  writeback(buf_a, i_out, 0)
  drain(i_out, half, buf_b, sem_b)
  writeback(buf_b, i_out, half)


@functools.cache
def _make_sc_gather():
  return pl.kernel(
      _sc_gather_body,
      out_type=[jax.ShapeDtypeStruct((NW, NCHUNK, CHUNK, 2 * D), jnp.float32)] * 2,
      mesh=plsc.VectorSubcoreMesh(core_axis_name="c", subcore_axis_name="s"),
      scratch_types=[
          pltpu.VMEM((NCHUNK, CHUNK), jnp.int32),
          pltpu.VMEM((NCHUNK, CHUNK), jnp.int32),
          pltpu.VMEM((NCHUNK // 2, CHUNK, 2 * D), jnp.float32),
          pltpu.VMEM((NCHUNK // 2, CHUNK, 2 * D), jnp.float32),
          pltpu.SemaphoreType.DMA,
          pltpu.SemaphoreType.DMA,
      ],
      compiler_params=pltpu.CompilerParams(use_tc_tiling_on_sc=False),
  )


def _tc_body(u_ref, i_ref, par_ref, w0u_ref, w0i_ref, b0_ref,
             w1_ref, b1_ref, w2_ref, b2_ref, wog_ref, woh_ref, bo_ref,
             out_ref):
  mm = lambda x, w: jax.lax.dot_general(
      x.astype(jnp.bfloat16), w.astype(jnp.bfloat16),
      (((1,), (0,)), ((), ())), preferred_element_type=jnp.float32)
  par = par_ref[...].astype(jnp.int32)

  def dec(r, shift):
    w = jax.lax.bitcast_convert_type(r[...], jnp.int32)
    return jax.lax.bitcast_convert_type((w >> shift) << 16, jnp.float32)

  u = dec(u_ref, (par & 1) * 16)
  it = dec(i_ref, ((par >> 1) & 1) * 16)
  g = u[:, :D] * it[:, :D]
  h = mm(u[:, D:], w0u_ref[...]) + mm(it[:, D:], w0i_ref[...]) + b0_ref[...]
  h = jnp.maximum(h, 0.0)
  h = jnp.maximum(mm(h, w1_ref[...]) + b1_ref[...], 0.0)
  h = jnp.maximum(mm(h, w2_ref[...]) + b2_ref[...], 0.0)
  logit = (
      jnp.sum(g * wog_ref[...], axis=1)
      + jnp.sum(h * woh_ref[...], axis=1)
      + bo_ref[0, 0]
  )
  out_ref[...] = 1.0 / (1.0 + jnp.exp(-logit))


def kernel(user_indices, item_indices, ug_table, ig_table, um_table, im_table,
           w0, b0, w1, b1, w2, b2, wo, bo):
  def rrow(ix):
    ix = ix.astype(jnp.int32)
    return (ix // PACK_N) * PACK_N2 + (ix % PACK_N) // 2

  uidx3 = rrow(user_indices).reshape(NW, NCHUNK, CHUNK)
  iidx3 = rrow(item_indices).reshape(NW, NCHUNK, CHUNK)
  par = (user_indices % 2 + 2 * (item_indices % 2)).astype(jnp.int8)
  par = par.reshape(B, 1)

  ugT, igT, umT, imT = (t.T for t in (ug_table, ig_table, um_table, im_table))

  user_pack, item_pack = _pack_tables(ugT, umT, igT, imT)

  u_rows, i_rows = _make_sc_gather()(uidx3, iidx3, user_pack, item_pack)
  u_rows = u_rows.reshape(B, 2 * D)
  i_rows = i_rows.reshape(B, 2 * D)

  w0u = w0[:, :D].T
  w0i = w0[:, D:].T
  w1t = w1.T
  w2t = w2.T
  wog = wo[:, :D]
  woh = wo[:, D:]
  b0r = b0.reshape(1, -1)
  b1r = b1.reshape(1, -1)
  b2r = b2.reshape(1, -1)
  bor = bo.reshape(1, 1)

  bt = 4096
  ntiles = B // bt
  full = lambda shape: pl.BlockSpec(shape, lambda i: (0, 0))
  out = pl.pallas_call(
      _tc_body,
      grid=(ntiles,),
      in_specs=[
          pl.BlockSpec((bt, 2 * D), lambda i: (i, 0)),
          pl.BlockSpec((bt, 2 * D), lambda i: (i, 0)),
          pl.BlockSpec((bt, 1), lambda i: (i, 0)),
          full((D, 128)),
          full((D, 128)),
          full((1, 128)),
          full((128, 64)),
          full((1, 64)),
          full((64, 32)),
          full((1, 32)),
          full((1, D)),
          full((1, 32)),
          full((1, 1)),
      ],
      out_specs=pl.BlockSpec((bt,), lambda i: (i,)),
      out_shape=jax.ShapeDtypeStruct((B,), jnp.float32),
  )(u_rows, i_rows, par, w0u, w0i, b0r, w1t, b1r, w2t, b2r, wog, woh, bor)
  return out
